# Initial kernel scaffold; baseline (speedup 1.0000x reference)
import jax
import jax.numpy as jnp
from jax.experimental import pallas as pl


def kernel(tokens, notes, durations, token_table, note_table, W, b):
    raise NotImplementedError("write your pallas kernel here")


# speedup vs baseline: 1.8490x; 1.8490x over previous
import functools

import jax
import jax.numpy as jnp
from jax import lax
from jax.experimental import pallas as pl
from jax.experimental.pallas import tpu as pltpu
from jax.experimental.pallas import tpu_sc as plsc

_NC = 2
_NS = 16
_NW = _NC * _NS
_LANES = 16


def _proj_body(x_ref, w_ref, o_ref):
    o_ref[...] = jnp.dot(x_ref[...], w_ref[...],
                         preferred_element_type=jnp.float32,
                         precision=jax.lax.Precision.HIGHEST)


def _proj_bias_body(x_ref, w_ref, b_ref, o_ref):
    o_ref[...] = jnp.dot(x_ref[...], w_ref[...],
                         preferred_element_type=jnp.float32,
                         precision=jax.lax.Precision.HIGHEST) + b_ref[...]


def _project_token_table(token_table, w1t):
    tokens_v, tok_sz = token_table.shape
    enc = w1t.shape[1]
    rows = 1000
    assert tokens_v % rows == 0
    grid = (tokens_v // rows,)
    return pl.pallas_call(
        _proj_body,
        grid=grid,
        in_specs=[
            pl.BlockSpec((rows, tok_sz), lambda i: (i, 0)),
            pl.BlockSpec((tok_sz, enc), lambda i: (0, 0)),
        ],
        out_specs=pl.BlockSpec((rows, enc), lambda i: (i, 0)),
        out_shape=jax.ShapeDtypeStruct((tokens_v, enc), jnp.float32),
    )(token_table, w1t)


def _project_note_table(note_table, w2t, b):
    max_note, note_sz = note_table.shape
    enc = w2t.shape[1]
    return pl.pallas_call(
        _proj_bias_body,
        out_shape=jax.ShapeDtypeStruct((max_note, enc), jnp.float32),
    )(note_table, w2t, b[None, :])


def _make_sc_gather_add(n, enc, chunk):
    assert n % (_NW * chunk) == 0
    per_w = n // _NW
    nchunks = per_w // chunk
    mesh = plsc.VectorSubcoreMesh(core_axis_name="c", subcore_axis_name="s")

    @functools.partial(
        pl.kernel,
        out_type=jax.ShapeDtypeStruct((n, enc), jnp.float32),
        mesh=mesh,
        compiler_params=pltpu.CompilerParams(use_tc_tiling_on_sc=False),
        scratch_types=[
            pltpu.VMEM((chunk,), jnp.int32),
            pltpu.VMEM((chunk,), jnp.int32),
            pltpu.VMEM((chunk, enc), jnp.float32),
            pltpu.VMEM((chunk, enc), jnp.float32),
            pltpu.SemaphoreType.DMA,
        ],
    )
    def sc_kernel(tp_hbm, np_hbm, tok_hbm, note_hbm, out_hbm,
                  tok_v, note_v, rows_a, rows_b, sem):
        wid = lax.axis_index("s") * _NC + lax.axis_index("c")
        base = wid * per_w

        def chunk_body(ci, carry):
            off = base + ci * chunk
            pltpu.sync_copy(tok_hbm.at[pl.ds(off, chunk)], tok_v)
            pltpu.sync_copy(note_hbm.at[pl.ds(off, chunk)], note_v)
            pltpu.async_copy(tp_hbm.at[tok_v], rows_a, sem).wait()
            pltpu.async_copy(np_hbm.at[note_v], rows_b, sem).wait()

            def row_body(r, rcarry):
                for j in range(enc // _LANES):
                    sl = pl.ds(j * _LANES, _LANES)
                    rows_a[r, sl] = rows_a[r, sl] + rows_b[r, sl]
                return rcarry

            lax.fori_loop(0, chunk, row_body, 0)
            pltpu.sync_copy(rows_a, out_hbm.at[pl.ds(off, chunk)])
            return carry

        lax.fori_loop(0, nchunks, chunk_body, 0)

    return sc_kernel


def kernel(tokens, notes, durations, token_table, note_table, W, b):
    bsz, t = tokens.shape
    tok_sz = token_table.shape[1]
    enc = W.shape[0]
    n = bsz * t

    w1t = W[:, :tok_sz].T
    w2t = W[:, tok_sz:].T

    tp = _project_token_table(token_table, w1t)
    npj = _project_note_table(note_table, w2t, b)

    tok_flat = tokens.reshape(-1).astype(jnp.int32)
    note_flat = notes.reshape(-1).astype(jnp.int32)

    out = _make_sc_gather_add(n, enc, 128)(tp, npj, tok_flat, note_flat)
    return out.reshape(bsz, t, enc)


# speedup vs baseline: 2.0608x; 1.1145x over previous
import functools

import jax
import jax.numpy as jnp
from jax import lax
from jax.experimental import pallas as pl
from jax.experimental.pallas import tpu as pltpu
from jax.experimental.pallas import tpu_sc as plsc

_NC = 2
_NS = 16
_NW = _NC * _NS
_LANES = 16
_CHUNK = 64


def _make_sc_gather(n, tok_sz, note_sz):
    per_w = n // _NW
    nchunks = per_w // _CHUNK
    assert per_w % _CHUNK == 0
    mesh = plsc.VectorSubcoreMesh(core_axis_name="c", subcore_axis_name="s")

    @functools.partial(
        pl.kernel,
        out_type=(
            jax.ShapeDtypeStruct((n, tok_sz), jnp.float32),
            jax.ShapeDtypeStruct((n, tok_sz), jnp.float32),
        ),
        mesh=mesh,
        compiler_params=pltpu.CompilerParams(use_tc_tiling_on_sc=False),
        scratch_types=[
            pltpu.VMEM((nchunks, _CHUNK), jnp.int32),
            pltpu.VMEM((nchunks, _CHUNK), jnp.int32),
            pltpu.VMEM((_CHUNK, tok_sz), jnp.float32),
            pltpu.VMEM((_CHUNK, note_sz), jnp.float32),
            pltpu.VMEM((_CHUNK, tok_sz), jnp.float32),
            pltpu.SemaphoreType.DMA,
            pltpu.SemaphoreType.DMA,
        ],
    )
    def sc_kernel(tok_tab, note_tab, tok_idx, note_idx, xa_out, xb_out,
                  tok_v, note_v, abuf, bsrc, bstg, sema, semb):
        wid = lax.axis_index("s") * _NC + lax.axis_index("c")
        base = wid * per_w

        pltpu.sync_copy(tok_idx.at[pl.ds(wid * nchunks, nchunks)], tok_v)
        pltpu.sync_copy(note_idx.at[pl.ds(wid * nchunks, nchunks)], note_v)

        zeros = jnp.zeros((_LANES,), jnp.float32)

        def zrow(r, carry):
            for k in range(note_sz, tok_sz, _LANES):
                bstg[r, pl.ds(k, _LANES)] = zeros
            return carry

        lax.fori_loop(0, _CHUNK, zrow, 0)

        def chunk_body(ci, carry):
            off = base + ci * _CHUNK
            pltpu.async_copy(tok_tab.at[tok_v.at[ci]], abuf, sema).wait()
            pltpu.async_copy(note_tab.at[note_v.at[ci]], bsrc, semb).wait()

            def crow(r, rc):
                for k in range(0, note_sz, _LANES):
                    bstg[r, pl.ds(k, _LANES)] = bsrc[r, pl.ds(k, _LANES)]
                return rc

            lax.fori_loop(0, _CHUNK, crow, 0)
            pltpu.sync_copy(abuf, xa_out.at[pl.ds(off, _CHUNK)])
            pltpu.sync_copy(bstg, xb_out.at[pl.ds(off, _CHUNK)])
            return carry

        lax.fori_loop(0, nchunks, chunk_body, 0)

    return sc_kernel


def _mm_body(xa_ref, xb_ref, w1_ref, w2_ref, b_ref, o_ref):
    acc = jnp.dot(xa_ref[...], w1_ref[...],
                  preferred_element_type=jnp.float32,
                  precision=jax.lax.Precision.HIGHEST)
    acc += jnp.dot(xb_ref[...], w2_ref[...],
                   preferred_element_type=jnp.float32,
                   precision=jax.lax.Precision.HIGHEST)
    o_ref[...] = acc + b_ref[...]


def _tc_matmul(xa, xb, w1t, w2tp, b):
    n, tok_sz = xa.shape
    enc = w1t.shape[1]
    rows = 2048
    assert n % rows == 0
    grid = (n // rows,)
    return pl.pallas_call(
        _mm_body,
        grid=grid,
        in_specs=[
            pl.BlockSpec((rows, tok_sz), lambda i: (i, 0)),
            pl.BlockSpec((rows, tok_sz), lambda i: (i, 0)),
            pl.BlockSpec((tok_sz, enc), lambda i: (0, 0)),
            pl.BlockSpec((tok_sz, enc), lambda i: (0, 0)),
            pl.BlockSpec((1, enc), lambda i: (0, 0)),
        ],
        out_specs=pl.BlockSpec((rows, enc), lambda i: (i, 0)),
        out_shape=jax.ShapeDtypeStruct((n, enc), jnp.float32),
    )(xa, xb, w1t, w2tp, b[None, :])


def kernel(tokens, notes, durations, token_table, note_table, W, b):
    bsz, t = tokens.shape
    tok_sz = token_table.shape[1]
    note_sz = note_table.shape[1]
    enc = W.shape[0]
    n = bsz * t

    w1t = W[:, :tok_sz].T
    w2tp = jnp.zeros((tok_sz, enc), jnp.float32).at[:note_sz].set(W[:, tok_sz:].T)

    tok_idx = tokens.reshape(n // _CHUNK, _CHUNK).astype(jnp.int32)
    note_idx = notes.reshape(n // _CHUNK, _CHUNK).astype(jnp.int32)

    xa, xb = _make_sc_gather(n, tok_sz, note_sz)(
        token_table, note_table, tok_idx, note_idx)
    y = _tc_matmul(xa, xb, w1t, w2tp, b)
    return y.reshape(bsz, t, enc)


# speedup vs baseline: 2.5197x; 1.2227x over previous
import functools

import jax
import jax.numpy as jnp
from jax import lax
from jax.experimental import pallas as pl
from jax.experimental.pallas import tpu as pltpu
from jax.experimental.pallas import tpu_sc as plsc

_NC = 2
_NS = 16
_NW = _NC * _NS
_LANES = 16
_CHUNK = 64


def _make_sc_gather(n, tok_sz, note_sz):
    per_w = n // _NW
    nchunks = per_w // _CHUNK
    assert per_w % _CHUNK == 0
    mesh = plsc.VectorSubcoreMesh(core_axis_name="c", subcore_axis_name="s")

    @functools.partial(
        pl.kernel,
        out_type=(
            jax.ShapeDtypeStruct((n, tok_sz), jnp.float32),
            jax.ShapeDtypeStruct((n, tok_sz), jnp.float32),
        ),
        mesh=mesh,
        compiler_params=pltpu.CompilerParams(use_tc_tiling_on_sc=False),
        scratch_types=[
            pltpu.VMEM((nchunks, _CHUNK), jnp.int32),
            pltpu.VMEM((nchunks, _CHUNK), jnp.int32),
            pltpu.VMEM((_CHUNK, tok_sz), jnp.float32),
            pltpu.VMEM((_CHUNK, note_sz), jnp.float32),
            pltpu.VMEM((_CHUNK, tok_sz), jnp.float32),
            pltpu.SemaphoreType.DMA,
            pltpu.SemaphoreType.DMA,
        ],
    )
    def sc_kernel(tok_tab, note_tab, tok_idx, note_idx, xa_out, xb_out,
                  tok_v, note_v, abuf, bsrc, bstg, sema, semb):
        wid = lax.axis_index("s") * _NC + lax.axis_index("c")
        base = wid * per_w

        pltpu.sync_copy(tok_idx.at[pl.ds(wid * nchunks, nchunks)], tok_v)
        pltpu.sync_copy(note_idx.at[pl.ds(wid * nchunks, nchunks)], note_v)

        zeros = jnp.zeros((_LANES,), jnp.float32)

        def zrow(r, carry):
            for k in range(note_sz, tok_sz, _LANES):
                bstg[r, pl.ds(k, _LANES)] = zeros
            return carry

        lax.fori_loop(0, _CHUNK, zrow, 0)

        def chunk_body(ci, carry):
            off = base + ci * _CHUNK
            pltpu.async_copy(tok_tab.at[tok_v.at[ci]], abuf, sema).wait()
            pltpu.async_copy(note_tab.at[note_v.at[ci]], bsrc, semb).wait()

            def crow(r, rc):
                for k in range(0, note_sz, _LANES):
                    bstg[r, pl.ds(k, _LANES)] = bsrc[r, pl.ds(k, _LANES)]
                return rc

            lax.fori_loop(0, _CHUNK, crow, 0)
            pltpu.sync_copy(abuf, xa_out.at[pl.ds(off, _CHUNK)])
            pltpu.sync_copy(bstg, xb_out.at[pl.ds(off, _CHUNK)])
            return carry

        lax.fori_loop(0, nchunks, chunk_body, 0)

    return sc_kernel


def _mm_body(xa_ref, xb_ref, w1_ref, w2_ref, b_ref, o_ref):
    bb, t, enc = o_ref.shape
    acc = jnp.dot(xa_ref[...], w1_ref[...],
                  preferred_element_type=jnp.float32,
                  precision=jax.lax.Precision.DEFAULT)
    acc += jnp.dot(xb_ref[...], w2_ref[...],
                   preferred_element_type=jnp.float32,
                   precision=jax.lax.Precision.DEFAULT)
    o_ref[...] = (acc + b_ref[...]).reshape(bb, t, enc)


def _tc_matmul(xa, xb, w1t, w2tp, b, bsz, t):
    n, tok_sz = xa.shape
    enc = w1t.shape[1]
    bblk = 16
    assert bsz % bblk == 0
    rows = bblk * t
    grid = (bsz // bblk,)
    return pl.pallas_call(
        _mm_body,
        grid=grid,
        in_specs=[
            pl.BlockSpec((rows, tok_sz), lambda i: (i, 0)),
            pl.BlockSpec((rows, tok_sz), lambda i: (i, 0)),
            pl.BlockSpec((tok_sz, enc), lambda i: (0, 0)),
            pl.BlockSpec((tok_sz, enc), lambda i: (0, 0)),
            pl.BlockSpec((1, enc), lambda i: (0, 0)),
        ],
        out_specs=pl.BlockSpec((bblk, t, enc), lambda i: (i, 0, 0)),
        out_shape=jax.ShapeDtypeStruct((bsz, t, enc), jnp.float32),
    )(xa, xb, w1t, w2tp, b[None, :])


def kernel(tokens, notes, durations, token_table, note_table, W, b):
    bsz, t = tokens.shape
    tok_sz = token_table.shape[1]
    note_sz = note_table.shape[1]
    enc = W.shape[0]
    n = bsz * t

    w1t = W[:, :tok_sz].T
    w2tp = jnp.zeros((tok_sz, enc), jnp.float32).at[:note_sz].set(W[:, tok_sz:].T)

    tok_idx = tokens.reshape(n // _CHUNK, _CHUNK).astype(jnp.int32)
    note_idx = notes.reshape(n // _CHUNK, _CHUNK).astype(jnp.int32)

    xa, xb = _make_sc_gather(n, tok_sz, note_sz)(
        token_table, note_table, tok_idx, note_idx)
    return _tc_matmul(xa, xb, w1t, w2tp, b, bsz, t)


# speedup vs baseline: 3.2189x; 1.2775x over previous
import functools

import jax
import jax.numpy as jnp
from jax import lax
from jax.experimental import pallas as pl
from jax.experimental.pallas import tpu as pltpu
from jax.experimental.pallas import tpu_sc as plsc

_NC = 2
_NS = 16
_NW = _NC * _NS
_LANES = 16
_CHUNK = 64
_NSLOT = 4
_LOOKAHEAD = 2


def _make_sc_gather(n, tok_sz, note_sz):
    per_w = n // _NW
    nchunks = per_w // _CHUNK
    assert per_w % _CHUNK == 0 and nchunks % _NSLOT == 0
    ntrips = nchunks // _NSLOT
    mesh = plsc.VectorSubcoreMesh(core_axis_name="c", subcore_axis_name="s")

    @functools.partial(
        pl.kernel,
        out_type=(
            jax.ShapeDtypeStruct((n, tok_sz), jnp.float32),
            jax.ShapeDtypeStruct((n, tok_sz), jnp.float32),
        ),
        mesh=mesh,
        compiler_params=pltpu.CompilerParams(use_tc_tiling_on_sc=False),
        scratch_types=[
            pltpu.VMEM((nchunks, _CHUNK), jnp.int32),
            pltpu.VMEM((nchunks, _CHUNK), jnp.int32),
        ] + [pltpu.VMEM((_CHUNK, tok_sz), jnp.float32) for _ in range(_NSLOT)]
          + [pltpu.VMEM((_CHUNK, note_sz), jnp.float32) for _ in range(_NSLOT)]
          + [pltpu.VMEM((_CHUNK, tok_sz), jnp.float32) for _ in range(_NSLOT)]
          + [
            pltpu.SemaphoreType.DMA,
            pltpu.SemaphoreType.DMA,
            pltpu.SemaphoreType.DMA,
            pltpu.SemaphoreType.DMA,
        ],
    )
    def sc_kernel(tok_tab, note_tab, tok_idx, note_idx, xa_out, xb_out,
                  tok_v, note_v, a0, a1, a2, a3, s0, s1, s2, s3,
                  g0, g1, g2, g3, sga, sgb, soa, sob):
        abuf = (a0, a1, a2, a3)
        bsrc = (s0, s1, s2, s3)
        bstg = (g0, g1, g2, g3)
        wid = lax.axis_index("s") * _NC + lax.axis_index("c")
        base = wid * per_w

        pltpu.sync_copy(tok_idx.at[pl.ds(wid * nchunks, nchunks)], tok_v)
        pltpu.sync_copy(note_idx.at[pl.ds(wid * nchunks, nchunks)], note_v)

        zeros = jnp.zeros((_LANES,), jnp.float32)

        def zrow(r, carry):
            for sl in range(_NSLOT):
                for k in range(note_sz, tok_sz, _LANES):
                    bstg[sl][r, pl.ds(k, _LANES)] = zeros
            return carry

        lax.fori_loop(0, _CHUNK, zrow, 0)

        def issue_gathers(c, sl):
            pltpu.async_copy(tok_tab.at[tok_v.at[c]], abuf[sl], sga)
            pltpu.async_copy(note_tab.at[note_v.at[c]], bsrc[sl], sgb)

        for sl in range(_LOOKAHEAD):
            issue_gathers(sl, sl)

        def trip(t, carry):
            for sl in range(_NSLOT):
                c = t * _NSLOT + sl
                pltpu.make_async_copy(tok_tab.at[tok_v.at[c]], abuf[sl], sga).wait()
                pltpu.make_async_copy(note_tab.at[note_v.at[c]], bsrc[sl], sgb).wait()
                @pl.when(c >= _LOOKAHEAD)
                def _retire():
                    cp = c - _LOOKAHEAD
                    slp = (sl + _NSLOT - _LOOKAHEAD) % _NSLOT
                    pltpu.make_async_copy(
                        abuf[slp], xa_out.at[pl.ds(base + cp * _CHUNK, _CHUNK)],
                        soa).wait()
                    pltpu.make_async_copy(
                        bstg[slp], xb_out.at[pl.ds(base + cp * _CHUNK, _CHUNK)],
                        sob).wait()

                def crow(r, rc):
                    for k in range(0, note_sz, _LANES):
                        bstg[sl][r, pl.ds(k, _LANES)] = bsrc[sl][r, pl.ds(k, _LANES)]
                    return rc

                lax.fori_loop(0, _CHUNK, crow, 0)

                @pl.when(c + _LOOKAHEAD < nchunks)
                def _refill():
                    issue_gathers(c + _LOOKAHEAD, (sl + _LOOKAHEAD) % _NSLOT)

                off = base + c * _CHUNK
                pltpu.async_copy(abuf[sl], xa_out.at[pl.ds(off, _CHUNK)], soa)
                pltpu.async_copy(bstg[sl], xb_out.at[pl.ds(off, _CHUNK)], sob)
            return carry

        lax.fori_loop(0, ntrips, trip, 0)

        for k in range(_LOOKAHEAD):
            cp = nchunks - _LOOKAHEAD + k
            slp = cp % _NSLOT
            pltpu.make_async_copy(
                abuf[slp], xa_out.at[pl.ds(base + cp * _CHUNK, _CHUNK)], soa).wait()
            pltpu.make_async_copy(
                bstg[slp], xb_out.at[pl.ds(base + cp * _CHUNK, _CHUNK)], sob).wait()

    return sc_kernel


def _mm_body(xa_ref, xb_ref, w1_ref, w2_ref, b_ref, o_ref):
    acc = jnp.dot(xa_ref[...], w1_ref[...], preferred_element_type=jnp.float32)
    acc += jnp.dot(xb_ref[...], w2_ref[...], preferred_element_type=jnp.float32)
    o_ref[...] = acc + b_ref[...]


def _tc_matmul(xa, xb, w1t, w2tp, b):
    n, tok_sz = xa.shape
    enc = w1t.shape[1]
    rows = 3200
    assert n % rows == 0
    grid = (n // rows,)
    return pl.pallas_call(
        _mm_body,
        grid=grid,
        in_specs=[
            pl.BlockSpec((rows, tok_sz), lambda i: (i, 0)),
            pl.BlockSpec((rows, tok_sz), lambda i: (i, 0)),
            pl.BlockSpec((tok_sz, enc), lambda i: (0, 0)),
            pl.BlockSpec((tok_sz, enc), lambda i: (0, 0)),
            pl.BlockSpec((1, enc), lambda i: (0, 0)),
        ],
        out_specs=pl.BlockSpec((rows, enc), lambda i: (i, 0)),
        out_shape=jax.ShapeDtypeStruct((n, enc), jnp.float32),
    )(xa, xb, w1t, w2tp, b[None, :])


def kernel(tokens, notes, durations, token_table, note_table, W, b):
    bsz, t = tokens.shape
    tok_sz = token_table.shape[1]
    note_sz = note_table.shape[1]
    enc = W.shape[0]
    n = bsz * t

    w1t = W[:, :tok_sz].T
    w2tp = jnp.zeros((tok_sz, enc), jnp.float32).at[:note_sz].set(W[:, tok_sz:].T)

    tok_idx = tokens.reshape(n // _CHUNK, _CHUNK).astype(jnp.int32)
    note_idx = notes.reshape(n // _CHUNK, _CHUNK).astype(jnp.int32)

    xa, xb = _make_sc_gather(n, tok_sz, note_sz)(
        token_table, note_table, tok_idx, note_idx)
    y = _tc_matmul(xa, xb, w1t, w2tp, b)
    return y.reshape(bsz, t, enc)


# speedup vs baseline: 3.2210x; 1.0006x over previous
import functools

import jax
import jax.numpy as jnp
from jax import lax
from jax.experimental import pallas as pl
from jax.experimental.pallas import tpu as pltpu
from jax.experimental.pallas import tpu_sc as plsc

_NC = 2
_NS = 16
_NW = _NC * _NS
_LANES = 16
_CHUNK = 80
_NSLOT = 4
_LOOKAHEAD = 2


def _make_sc_gather(n, tok_sz, note_sz):
    per_w = n // _NW
    nchunks = per_w // _CHUNK
    assert per_w % _CHUNK == 0 and nchunks % _NSLOT == 0
    ntrips = nchunks // _NSLOT
    mesh = plsc.VectorSubcoreMesh(core_axis_name="c", subcore_axis_name="s")

    @functools.partial(
        pl.kernel,
        out_type=(
            jax.ShapeDtypeStruct((n, tok_sz), jnp.float32),
            jax.ShapeDtypeStruct((n, tok_sz), jnp.float32),
        ),
        mesh=mesh,
        compiler_params=pltpu.CompilerParams(use_tc_tiling_on_sc=False),
        scratch_types=[
            pltpu.VMEM((nchunks, _CHUNK), jnp.int32),
            pltpu.VMEM((nchunks, _CHUNK), jnp.int32),
        ] + [pltpu.VMEM((_CHUNK, tok_sz), jnp.float32) for _ in range(_NSLOT)]
          + [pltpu.VMEM((_CHUNK, note_sz), jnp.float32) for _ in range(_NSLOT)]
          + [pltpu.VMEM((_CHUNK, tok_sz), jnp.float32) for _ in range(_NSLOT)]
          + [
            pltpu.SemaphoreType.DMA,
            pltpu.SemaphoreType.DMA,
            pltpu.SemaphoreType.DMA,
            pltpu.SemaphoreType.DMA,
        ],
    )
    def sc_kernel(tok_tab, note_tab, tok_idx, note_idx, xa_out, xb_out,
                  tok_v, note_v, a0, a1, a2, a3, s0, s1, s2, s3,
                  g0, g1, g2, g3, sga, sgb, soa, sob):
        abuf = (a0, a1, a2, a3)
        bsrc = (s0, s1, s2, s3)
        bstg = (g0, g1, g2, g3)
        wid = lax.axis_index("s") * _NC + lax.axis_index("c")
        base = wid * per_w

        pltpu.sync_copy(tok_idx.at[pl.ds(wid * nchunks, nchunks)], tok_v)
        pltpu.sync_copy(note_idx.at[pl.ds(wid * nchunks, nchunks)], note_v)

        zeros = jnp.zeros((_LANES,), jnp.float32)

        def zrow(r, carry):
            for sl in range(_NSLOT):
                for k in range(note_sz, tok_sz, _LANES):
                    bstg[sl][r, pl.ds(k, _LANES)] = zeros
            return carry

        lax.fori_loop(0, _CHUNK, zrow, 0)

        def issue_gathers(c, sl):
            pltpu.async_copy(tok_tab.at[tok_v.at[c]], abuf[sl], sga)
            pltpu.async_copy(note_tab.at[note_v.at[c]], bsrc[sl], sgb)

        for sl in range(_LOOKAHEAD):
            issue_gathers(sl, sl)

        def trip(t, carry):
            for sl in range(_NSLOT):
                c = t * _NSLOT + sl
                pltpu.make_async_copy(tok_tab.at[tok_v.at[c]], abuf[sl], sga).wait()
                pltpu.make_async_copy(note_tab.at[note_v.at[c]], bsrc[sl], sgb).wait()
                @pl.when(c >= _LOOKAHEAD)
                def _retire():
                    cp = c - _LOOKAHEAD
                    slp = (sl + _NSLOT - _LOOKAHEAD) % _NSLOT
                    pltpu.make_async_copy(
                        abuf[slp], xa_out.at[pl.ds(base + cp * _CHUNK, _CHUNK)],
                        soa).wait()
                    pltpu.make_async_copy(
                        bstg[slp], xb_out.at[pl.ds(base + cp * _CHUNK, _CHUNK)],
                        sob).wait()

                def crow(r, rc):
                    for k in range(0, note_sz, _LANES):
                        bstg[sl][r, pl.ds(k, _LANES)] = bsrc[sl][r, pl.ds(k, _LANES)]
                    return rc

                lax.fori_loop(0, _CHUNK, crow, 0)

                @pl.when(c + _LOOKAHEAD < nchunks)
                def _refill():
                    issue_gathers(c + _LOOKAHEAD, (sl + _LOOKAHEAD) % _NSLOT)

                off = base + c * _CHUNK
                pltpu.async_copy(abuf[sl], xa_out.at[pl.ds(off, _CHUNK)], soa)
                pltpu.async_copy(bstg[sl], xb_out.at[pl.ds(off, _CHUNK)], sob)
            return carry

        lax.fori_loop(0, ntrips, trip, 0)

        for k in range(_LOOKAHEAD):
            cp = nchunks - _LOOKAHEAD + k
            slp = cp % _NSLOT
            pltpu.make_async_copy(
                abuf[slp], xa_out.at[pl.ds(base + cp * _CHUNK, _CHUNK)], soa).wait()
            pltpu.make_async_copy(
                bstg[slp], xb_out.at[pl.ds(base + cp * _CHUNK, _CHUNK)], sob).wait()

    return sc_kernel


def _mm_body(xa_ref, xb_ref, w1_ref, w2_ref, b_ref, o_ref):
    acc = jnp.dot(xa_ref[...], w1_ref[...], preferred_element_type=jnp.float32)
    acc += jnp.dot(xb_ref[...], w2_ref[...], preferred_element_type=jnp.float32)
    o_ref[...] = acc + b_ref[...]


def _tc_matmul(xa, xb, w1t, w2tp, b):
    n, tok_sz = xa.shape
    enc = w1t.shape[1]
    rows = 3200
    assert n % rows == 0
    grid = (n // rows,)
    return pl.pallas_call(
        _mm_body,
        grid=grid,
        in_specs=[
            pl.BlockSpec((rows, tok_sz), lambda i: (i, 0)),
            pl.BlockSpec((rows, tok_sz), lambda i: (i, 0)),
            pl.BlockSpec((tok_sz, enc), lambda i: (0, 0)),
            pl.BlockSpec((tok_sz, enc), lambda i: (0, 0)),
            pl.BlockSpec((1, enc), lambda i: (0, 0)),
        ],
        out_specs=pl.BlockSpec((rows, enc), lambda i: (i, 0)),
        out_shape=jax.ShapeDtypeStruct((n, enc), jnp.float32),
    )(xa, xb, w1t, w2tp, b[None, :])


def kernel(tokens, notes, durations, token_table, note_table, W, b):
    bsz, t = tokens.shape
    tok_sz = token_table.shape[1]
    note_sz = note_table.shape[1]
    enc = W.shape[0]
    n = bsz * t

    w1t = W[:, :tok_sz].T
    w2tp = jnp.zeros((tok_sz, enc), jnp.float32).at[:note_sz].set(W[:, tok_sz:].T)

    tok_idx = tokens.reshape(n // _CHUNK, _CHUNK).astype(jnp.int32)
    note_idx = notes.reshape(n // _CHUNK, _CHUNK).astype(jnp.int32)

    xa, xb = _make_sc_gather(n, tok_sz, note_sz)(
        token_table, note_table, tok_idx, note_idx)
    y = _tc_matmul(xa, xb, w1t, w2tp, b)
    return y.reshape(bsz, t, enc)


# speedup vs baseline: 3.2326x; 1.0036x over previous
import functools

import jax
import jax.numpy as jnp
from jax import lax
from jax.experimental import pallas as pl
from jax.experimental.pallas import tpu as pltpu
from jax.experimental.pallas import tpu_sc as plsc

_NC = 2
_NS = 16
_NW = _NC * _NS
_LANES = 16
_CHUNK = 40
_NSLOT = 8
_LOOKAHEAD = 4


def _make_sc_gather(n, tok_sz, note_sz):
    per_w = n // _NW
    nchunks = per_w // _CHUNK
    assert per_w % _CHUNK == 0 and nchunks % _NSLOT == 0
    ntrips = nchunks // _NSLOT
    mesh = plsc.VectorSubcoreMesh(core_axis_name="c", subcore_axis_name="s")

    @functools.partial(
        pl.kernel,
        out_type=(
            jax.ShapeDtypeStruct((n, tok_sz), jnp.float32),
            jax.ShapeDtypeStruct((n, tok_sz), jnp.float32),
        ),
        mesh=mesh,
        compiler_params=pltpu.CompilerParams(use_tc_tiling_on_sc=False),
        scratch_types=[
            pltpu.VMEM((nchunks, _CHUNK), jnp.int32),
            pltpu.VMEM((nchunks, _CHUNK), jnp.int32),
        ] + [pltpu.VMEM((_CHUNK, tok_sz), jnp.float32) for _ in range(_NSLOT)]
          + [pltpu.VMEM((_CHUNK, note_sz), jnp.float32) for _ in range(_NSLOT)]
          + [pltpu.VMEM((_CHUNK, tok_sz), jnp.float32) for _ in range(_NSLOT)]
          + [
            pltpu.SemaphoreType.DMA,
            pltpu.SemaphoreType.DMA,
            pltpu.SemaphoreType.DMA,
            pltpu.SemaphoreType.DMA,
        ],
    )
    def sc_kernel(tok_tab, note_tab, tok_idx, note_idx, xa_out, xb_out,
                  tok_v, note_v,
                  a0, a1, a2, a3, a4, a5, a6, a7,
                  s0, s1, s2, s3, s4, s5, s6, s7,
                  g0, g1, g2, g3, g4, g5, g6, g7,
                  sga, sgb, soa, sob):
        abuf = (a0, a1, a2, a3, a4, a5, a6, a7)
        bsrc = (s0, s1, s2, s3, s4, s5, s6, s7)
        bstg = (g0, g1, g2, g3, g4, g5, g6, g7)
        wid = lax.axis_index("s") * _NC + lax.axis_index("c")
        base = wid * per_w

        pltpu.sync_copy(tok_idx.at[pl.ds(wid * nchunks, nchunks)], tok_v)
        pltpu.sync_copy(note_idx.at[pl.ds(wid * nchunks, nchunks)], note_v)

        zeros = jnp.zeros((_LANES,), jnp.float32)

        def zrow(r, carry):
            for sl in range(_NSLOT):
                for k in range(note_sz, tok_sz, _LANES):
                    bstg[sl][r, pl.ds(k, _LANES)] = zeros
            return carry

        lax.fori_loop(0, _CHUNK, zrow, 0)

        def issue_gathers(c, sl):
            pltpu.async_copy(tok_tab.at[tok_v.at[c]], abuf[sl], sga)
            pltpu.async_copy(note_tab.at[note_v.at[c]], bsrc[sl], sgb)

        for sl in range(_LOOKAHEAD):
            issue_gathers(sl, sl)

        def trip(t, carry):
            for sl in range(_NSLOT):
                c = t * _NSLOT + sl
                pltpu.make_async_copy(tok_tab.at[tok_v.at[c]], abuf[sl], sga).wait()
                pltpu.make_async_copy(note_tab.at[note_v.at[c]], bsrc[sl], sgb).wait()
                @pl.when(c >= _LOOKAHEAD)
                def _retire():
                    cp = c - _LOOKAHEAD
                    slp = (sl + _NSLOT - _LOOKAHEAD) % _NSLOT
                    pltpu.make_async_copy(
                        abuf[slp], xa_out.at[pl.ds(base + cp * _CHUNK, _CHUNK)],
                        soa).wait()
                    pltpu.make_async_copy(
                        bstg[slp], xb_out.at[pl.ds(base + cp * _CHUNK, _CHUNK)],
                        sob).wait()

                def crow(r2, rc):
                    for dr in range(2):
                        r = r2 * 2 + dr
                        for k in range(0, note_sz, _LANES):
                            bstg[sl][r, pl.ds(k, _LANES)] = bsrc[sl][r, pl.ds(k, _LANES)]
                    return rc

                lax.fori_loop(0, _CHUNK // 2, crow, 0)

                @pl.when(c + _LOOKAHEAD < nchunks)
                def _refill():
                    issue_gathers(c + _LOOKAHEAD, (sl + _LOOKAHEAD) % _NSLOT)

                off = base + c * _CHUNK
                pltpu.async_copy(abuf[sl], xa_out.at[pl.ds(off, _CHUNK)], soa)
                pltpu.async_copy(bstg[sl], xb_out.at[pl.ds(off, _CHUNK)], sob)
            return carry

        lax.fori_loop(0, ntrips, trip, 0)

        for k in range(_LOOKAHEAD):
            cp = nchunks - _LOOKAHEAD + k
            slp = cp % _NSLOT
            pltpu.make_async_copy(
                abuf[slp], xa_out.at[pl.ds(base + cp * _CHUNK, _CHUNK)], soa).wait()
            pltpu.make_async_copy(
                bstg[slp], xb_out.at[pl.ds(base + cp * _CHUNK, _CHUNK)], sob).wait()

    return sc_kernel


def _mm_body(xa_ref, xb_ref, w1_ref, w2_ref, b_ref, o_ref):
    acc = jnp.dot(xa_ref[...], w1_ref[...], preferred_element_type=jnp.float32)
    acc += jnp.dot(xb_ref[...], w2_ref[...], preferred_element_type=jnp.float32)
    o_ref[...] = acc + b_ref[...]


def _tc_matmul(xa, xb, w1t, w2tp, b):
    n, tok_sz = xa.shape
    enc = w1t.shape[1]
    rows = 3200
    assert n % rows == 0
    grid = (n // rows,)
    return pl.pallas_call(
        _mm_body,
        grid=grid,
        in_specs=[
            pl.BlockSpec((rows, tok_sz), lambda i: (i, 0)),
            pl.BlockSpec((rows, tok_sz), lambda i: (i, 0)),
            pl.BlockSpec((tok_sz, enc), lambda i: (0, 0)),
            pl.BlockSpec((tok_sz, enc), lambda i: (0, 0)),
            pl.BlockSpec((1, enc), lambda i: (0, 0)),
        ],
        out_specs=pl.BlockSpec((rows, enc), lambda i: (i, 0)),
        out_shape=jax.ShapeDtypeStruct((n, enc), jnp.float32),
    )(xa, xb, w1t, w2tp, b[None, :])


def kernel(tokens, notes, durations, token_table, note_table, W, b):
    bsz, t = tokens.shape
    tok_sz = token_table.shape[1]
    note_sz = note_table.shape[1]
    enc = W.shape[0]
    n = bsz * t

    w1t = W[:, :tok_sz].T
    w2tp = jnp.zeros((tok_sz, enc), jnp.float32).at[:note_sz].set(W[:, tok_sz:].T)

    tok_idx = tokens.reshape(n // _CHUNK, _CHUNK).astype(jnp.int32)
    note_idx = notes.reshape(n // _CHUNK, _CHUNK).astype(jnp.int32)

    xa, xb = _make_sc_gather(n, tok_sz, note_sz)(
        token_table, note_table, tok_idx, note_idx)
    y = _tc_matmul(xa, xb, w1t, w2tp, b)
    return y.reshape(bsz, t, enc)


# speedup vs baseline: 3.3482x; 1.0358x over previous
import functools

import jax
import jax.numpy as jnp
from jax import lax
from jax.experimental import pallas as pl
from jax.experimental.pallas import tpu as pltpu
from jax.experimental.pallas import tpu_sc as plsc

_NC = 2
_NS = 16
_NW = _NC * _NS
_LANES = 16
_CHUNK = 40
_NSLOT = 8
_LOOKAHEAD = 4


def _make_sc_gather(n, tok_sz, note_sz):
    per_w = n // _NW
    nchunks = per_w // _CHUNK
    assert per_w % _CHUNK == 0 and nchunks % _NSLOT == 0
    ntrips = nchunks // _NSLOT
    mesh = plsc.VectorSubcoreMesh(core_axis_name="c", subcore_axis_name="s")

    @functools.partial(
        pl.kernel,
        out_type=(
            jax.ShapeDtypeStruct((n, tok_sz), jnp.float32),
            jax.ShapeDtypeStruct((n, tok_sz), jnp.float32),
        ),
        mesh=mesh,
        compiler_params=pltpu.CompilerParams(use_tc_tiling_on_sc=False),
        scratch_types=[
            pltpu.VMEM((nchunks, _CHUNK), jnp.int32),
            pltpu.VMEM((nchunks, _CHUNK), jnp.int32),
        ] + [pltpu.VMEM((_CHUNK, tok_sz), jnp.float32) for _ in range(_NSLOT)]
          + [pltpu.VMEM((_CHUNK, note_sz), jnp.float32) for _ in range(_NSLOT)]
          + [pltpu.VMEM((_CHUNK, tok_sz), jnp.float32) for _ in range(_NSLOT)]
          + [
            pltpu.SemaphoreType.DMA,
            pltpu.SemaphoreType.DMA,
            pltpu.SemaphoreType.DMA,
            pltpu.SemaphoreType.DMA,
        ],
    )
    def sc_kernel(tok_tab, note_tab, tok_idx, note_idx, xa_out, xb_out,
                  tok_v, note_v,
                  a0, a1, a2, a3, a4, a5, a6, a7,
                  s0, s1, s2, s3, s4, s5, s6, s7,
                  g0, g1, g2, g3, g4, g5, g6, g7,
                  sga, sgb, soa, sob):
        abuf = (a0, a1, a2, a3, a4, a5, a6, a7)
        bsrc = (s0, s1, s2, s3, s4, s5, s6, s7)
        bstg = (g0, g1, g2, g3, g4, g5, g6, g7)
        wid = lax.axis_index("s") * _NC + lax.axis_index("c")
        base = wid * per_w

        pltpu.sync_copy(tok_idx.at[pl.ds(wid * nchunks, nchunks)], tok_v)
        pltpu.sync_copy(note_idx.at[pl.ds(wid * nchunks, nchunks)], note_v)

        zeros = jnp.zeros((_LANES,), jnp.float32)

        def zrow(r, carry):
            for sl in range(_NSLOT):
                for k in range(note_sz, tok_sz, _LANES):
                    bstg[sl][r, pl.ds(k, _LANES)] = zeros
            return carry

        lax.fori_loop(0, _CHUNK, zrow, 0)

        def issue_gathers(c, sl):
            pltpu.async_copy(tok_tab.at[tok_v.at[c]], abuf[sl], sga)
            pltpu.async_copy(note_tab.at[note_v.at[c]], bsrc[sl], sgb)

        for sl in range(_LOOKAHEAD):
            issue_gathers(sl, sl)

        def trip(t, carry):
            for sl in range(_NSLOT):
                c = t * _NSLOT + sl
                pltpu.make_async_copy(tok_tab.at[tok_v.at[c]], abuf[sl], sga).wait()
                pltpu.make_async_copy(note_tab.at[note_v.at[c]], bsrc[sl], sgb).wait()
                @pl.when(c >= _LOOKAHEAD)
                def _retire():
                    cp = c - _LOOKAHEAD
                    slp = (sl + _NSLOT - _LOOKAHEAD) % _NSLOT
                    pltpu.make_async_copy(
                        abuf[slp], xa_out.at[pl.ds(base + cp * _CHUNK, _CHUNK)],
                        soa).wait()
                    pltpu.make_async_copy(
                        bstg[slp], xb_out.at[pl.ds(base + cp * _CHUNK, _CHUNK)],
                        sob).wait()

                def crow(r2, rc):
                    for dr in range(2):
                        r = r2 * 2 + dr
                        for k in range(0, note_sz, _LANES):
                            bstg[sl][r, pl.ds(k, _LANES)] = bsrc[sl][r, pl.ds(k, _LANES)]
                    return rc

                lax.fori_loop(0, _CHUNK // 2, crow, 0)

                @pl.when(c + _LOOKAHEAD < nchunks)
                def _refill():
                    issue_gathers(c + _LOOKAHEAD, (sl + _LOOKAHEAD) % _NSLOT)

                off = base + c * _CHUNK
                pltpu.async_copy(abuf[sl], xa_out.at[pl.ds(off, _CHUNK)], soa)
                pltpu.async_copy(bstg[sl], xb_out.at[pl.ds(off, _CHUNK)], sob)
            return carry

        lax.fori_loop(0, ntrips, trip, 0)

        for k in range(_LOOKAHEAD):
            cp = nchunks - _LOOKAHEAD + k
            slp = cp % _NSLOT
            pltpu.make_async_copy(
                abuf[slp], xa_out.at[pl.ds(base + cp * _CHUNK, _CHUNK)], soa).wait()
            pltpu.make_async_copy(
                bstg[slp], xb_out.at[pl.ds(base + cp * _CHUNK, _CHUNK)], sob).wait()

    return sc_kernel


def _mm_body(xa_ref, xb_ref, w1_ref, w2_ref, b_ref, o_ref):
    acc = jnp.dot(xa_ref[...], w1_ref[...], preferred_element_type=jnp.float32)
    acc += jnp.dot(xb_ref[...], w2_ref[...], preferred_element_type=jnp.float32)
    o_ref[...] = acc + b_ref[...]


def _mm_body2(y_ref, xa_ref, xb_ref, w1_ref, w2_ref, b_ref, o_ref):
    _mm_body(xa_ref, xb_ref, w1_ref, w2_ref, b_ref, o_ref)


def _tc_matmul_first(xa, xb, w1t, w2tp, b, n_total):
    n, tok_sz = xa.shape
    enc = w1t.shape[1]
    rows = 3200
    assert n % rows == 0
    grid = (n // rows,)
    return pl.pallas_call(
        _mm_body,
        grid=grid,
        in_specs=[
            pl.BlockSpec((rows, tok_sz), lambda i: (i, 0)),
            pl.BlockSpec((rows, tok_sz), lambda i: (i, 0)),
            pl.BlockSpec((tok_sz, enc), lambda i: (0, 0)),
            pl.BlockSpec((tok_sz, enc), lambda i: (0, 0)),
            pl.BlockSpec((1, enc), lambda i: (0, 0)),
        ],
        out_specs=pl.BlockSpec((rows, enc), lambda i: (i, 0)),
        out_shape=jax.ShapeDtypeStruct((n_total, enc), jnp.float32),
    )(xa, xb, w1t, w2tp, b[None, :])


def _tc_matmul_second(y_prev, xa, xb, w1t, w2tp, b, blk_off):
    n, tok_sz = xa.shape
    enc = w1t.shape[1]
    rows = 3200
    assert n % rows == 0
    grid = (n // rows,)
    return pl.pallas_call(
        _mm_body2,
        grid=grid,
        in_specs=[
            pl.BlockSpec(memory_space=pl.ANY),
            pl.BlockSpec((rows, tok_sz), lambda i: (i, 0)),
            pl.BlockSpec((rows, tok_sz), lambda i: (i, 0)),
            pl.BlockSpec((tok_sz, enc), lambda i: (0, 0)),
            pl.BlockSpec((tok_sz, enc), lambda i: (0, 0)),
            pl.BlockSpec((1, enc), lambda i: (0, 0)),
        ],
        out_specs=pl.BlockSpec((rows, enc), lambda i: (i + blk_off, 0)),
        out_shape=jax.ShapeDtypeStruct(y_prev.shape, jnp.float32),
        input_output_aliases={0: 0},
    )(y_prev, xa, xb, w1t, w2tp, b[None, :])


def kernel(tokens, notes, durations, token_table, note_table, W, b):
    bsz, t = tokens.shape
    tok_sz = token_table.shape[1]
    note_sz = note_table.shape[1]
    enc = W.shape[0]
    n = bsz * t

    w1t = W[:, :tok_sz].T
    w2tp = jnp.zeros((tok_sz, enc), jnp.float32).at[:note_sz].set(W[:, tok_sz:].T)

    tok_idx = tokens.reshape(n // _CHUNK, _CHUNK).astype(jnp.int32)
    note_idx = notes.reshape(n // _CHUNK, _CHUNK).astype(jnp.int32)

    nh = n // 2
    rh = nh // _CHUNK
    sc_gather = _make_sc_gather(nh, tok_sz, note_sz)
    xa1, xb1 = sc_gather(token_table, note_table,
                         tok_idx[:rh], note_idx[:rh])
    xa2, xb2 = sc_gather(token_table, note_table,
                         tok_idx[rh:], note_idx[rh:])
    y1 = _tc_matmul_first(xa1, xb1, w1t, w2tp, b, n)
    y = _tc_matmul_second(y1, xa2, xb2, w1t, w2tp, b, nh // 3200)
    return y.reshape(bsz, t, enc)
